# baseline (speedup 1.0000x reference)
import jax
import jax.numpy as jnp
from jax.experimental import pallas as pl
from jax.experimental.pallas import tpu as pltpu
from jax.experimental.pallas import tpu_sc as plsc

B, S, E, N1, N2 = 16, 256, 256, 8, 8
KV, QD = 100, 768
GW = 128
C1W = 256


def _prep_body(ient_ref, q0_ref, wq2t_ref, bq2_ref, wk2_ref, wq1t_ref,
               bq1_ref, wk1_ref, q2_ref, q1_ref, idx_ref):
    f32 = jnp.float32
    q0 = q0_ref[...]
    qi2 = jnp.tanh(jnp.dot(q0, wq2t_ref[...]) + bq2_ref[...])
    qk2 = jnp.dot(qi2, wk2_ref[...]) * 0.1
    qi1 = jnp.tanh(jnp.dot(q0, wq1t_ref[...]) + bq1_ref[...])
    qk1 = jnp.dot(qi1, wk1_ref[...]) * 0.1
    qk2t = jnp.transpose(qk2)
    qk1t = jnp.transpose(qk1)
    for b in range(B):
        q2_ref[b] = qk2t[:, b:b + 1]
        q1_ref[b] = qk1t[:, b:b + 1]

    ie = ient_ref[:, 0, :]
    mask = ie != 0
    mf = mask.astype(f32)
    t_iota = jax.lax.broadcasted_iota(jnp.int32, (S, S), 0)
    s_iota = jax.lax.broadcasted_iota(jnp.int32, (S, S), 1)
    ut = (t_iota <= s_iota).astype(f32)
    cum = jnp.dot(mf, ut)
    order = jnp.clip(cum - 1.0, 0.0, float(E - 1))
    bi = jax.lax.broadcasted_iota(jnp.int32, (B, S), 0).astype(f32)
    idxf = jnp.where(mask, order + bi * float(E), float(B * E))
    idx_ref[...] = idxf.astype(jnp.int32)


def _main_body(q2_ref, q1_ref, k2_ref, v2_ref, k1_ref, v1_ref, c1_ref):
    f32 = jnp.float32
    bidx = pl.program_id(0)

    @pl.when(bidx < B)
    def _compute():
        k2 = k2_ref[0]
        v2 = v2_ref[0]
        q2 = q2_ref[0][None, :, :, None]
        l2 = jnp.sum(k2 * q2, axis=1)
        e2 = jnp.exp(l2)
        attn2 = e2 / jnp.sum(e2, axis=1, keepdims=True)
        comb = jnp.sum(attn2[:, None, :, :] * v2, axis=2)

        k1 = k1_ref[0]
        v1 = v1_ref[0]
        q1 = q1_ref[0][:, :, None]
        l1 = jnp.sum(k1 * q1, axis=0)
        e1 = jnp.exp(l1)
        attn1 = e1 / jnp.sum(e1, axis=0, keepdims=True)
        o_a = jnp.sum(attn1[None, :, :] * v1, axis=1)
        o_b = jnp.sum(attn1[:, None, :] * comb, axis=0)
        c1 = jnp.concatenate([o_a, o_b], axis=0)
        c1t = jnp.transpose(c1)
        c1_ref[...] = jnp.pad(c1t, ((0, 0), (0, C1W - 2 * KV)))

    @pl.when(bidx == B)
    def _zeros():
        c1_ref[...] = jnp.zeros((E, C1W), f32)


def _sc_gather(c1r, idx):
    vmesh = plsc.VectorSubcoreMesh(core_axis_name="core",
                                   subcore_axis_name="subcore")

    @pl.kernel(out_type=jax.ShapeDtypeStruct((B * S, C1W), c1r.dtype),
               mesh=vmesh)
    def gather_kernel(c1_hbm, i_hbm, o_hbm):
        def body(i_vmem, o_vmem):
            pltpu.sync_copy(c1_hbm.at[i_vmem.at[0]], o_vmem)

        pltpu.emit_pipeline(
            body,
            grid=(B * S // GW,),
            in_specs=[pl.BlockSpec((1, GW), index_map=lambda i: (0, i))],
            out_specs=[pl.BlockSpec((GW, C1W), index_map=lambda i: (i, 0))],
            core_axis_name=("core", "subcore"),
            dimension_semantics=(pltpu.PARALLEL,),
        )(i_hbm, o_hbm)

    return gather_kernel(c1r, idx)


@jax.jit
def _run(input_ent, q, k_hop1, v_hop1, k_hop2, v_hop2, w_q2, b_q2, w_k2,
         w_q1, b_q1, w_k1):
    f32 = jnp.float32
    q0 = q[:, 0, :]
    ient = input_ent.astype(jnp.int32).reshape(B, 1, S)
    qc2, qc1, idx = pl.pallas_call(
        _prep_body,
        out_shape=(jax.ShapeDtypeStruct((B, KV, 1), f32),
                   jax.ShapeDtypeStruct((B, KV, 1), f32),
                   jax.ShapeDtypeStruct((B, S), jnp.int32)),
    )(ient, q0, w_q2.T, b_q2.reshape(1, KV), w_k2, w_q1.T,
      b_q1.reshape(1, KV), w_k1)

    k2t = jnp.transpose(k_hop2, (0, 2, 4, 3, 1))
    v2t = jnp.transpose(v_hop2, (0, 2, 4, 3, 1))
    k1t = jnp.transpose(k_hop1, (0, 3, 2, 1))
    v1t = jnp.transpose(v_hop1, (0, 3, 2, 1))

    cb = lambda b: jnp.minimum(b, B - 1)
    c1r = pl.pallas_call(
        _main_body,
        grid=(B + 1,),
        in_specs=[
            pl.BlockSpec((1, KV, 1), lambda b: (cb(b), 0, 0)),
            pl.BlockSpec((1, KV, 1), lambda b: (cb(b), 0, 0)),
            pl.BlockSpec((1, N1, KV, N2, E), lambda b: (cb(b), 0, 0, 0, 0)),
            pl.BlockSpec((1, N1, KV, N2, E), lambda b: (cb(b), 0, 0, 0, 0)),
            pl.BlockSpec((1, KV, N1, E), lambda b: (cb(b), 0, 0, 0)),
            pl.BlockSpec((1, KV, N1, E), lambda b: (cb(b), 0, 0, 0)),
        ],
        out_specs=pl.BlockSpec((E, C1W), lambda b: (b, 0)),
        out_shape=jax.ShapeDtypeStruct(((B + 1) * E, C1W), f32),
        compiler_params=pltpu.CompilerParams(
            dimension_semantics=("arbitrary",),
        ),
    )(qc2, qc1, k2t, v2t, k1t, v1t)

    out_flat = _sc_gather(c1r, idx.reshape(1, B * S))
    return out_flat[:, :2 * KV].reshape(B, S, 2 * KV)


def kernel(input_ent, q, k_hop1, v_hop1, k_hop2, v_hop2, w_q2, b_q2, w_k2,
           w_q1, b_q1, w_k1):
    return _run(input_ent, q, k_hop1, v_hop1, k_hop2, v_hop2, w_q2, b_q2,
                w_k2, w_q1, b_q1, w_k1)


# speedup vs baseline: 1.0764x; 1.0764x over previous
import jax
import jax.numpy as jnp
from jax.experimental import pallas as pl
from jax.experimental.pallas import tpu as pltpu
from jax.experimental.pallas import tpu_sc as plsc

B, S, E, N1, N2 = 16, 256, 256, 8, 8
KV, QD = 100, 768
GW = 128
C1W = 256


def _prep_body(ient_ref, q0_ref, wq2_ref, bq2_ref, wk2_ref, wq1_ref,
               bq1_ref, wk1_ref, q2_ref, q1_ref, idx_ref):
    f32 = jnp.float32
    dn = (((1,), (1,)), ((), ()))
    q0 = q0_ref[...]
    qi2 = jnp.tanh(jax.lax.dot_general(q0, wq2_ref[...], dn) + bq2_ref[...])
    qk2 = jnp.dot(qi2, wk2_ref[...]) * 0.1
    qi1 = jnp.tanh(jax.lax.dot_general(q0, wq1_ref[...], dn) + bq1_ref[...])
    qk1 = jnp.dot(qi1, wk1_ref[...]) * 0.1
    qk2t = jnp.transpose(qk2)
    qk1t = jnp.transpose(qk1)
    for b in range(B):
        q2_ref[b] = qk2t[:, b:b + 1]
        q1_ref[b] = qk1t[:, b:b + 1]

    ie = ient_ref[...]
    mask = ie != 0
    mf = mask.astype(f32)
    t_iota = jax.lax.broadcasted_iota(jnp.int32, (S, S), 0)
    s_iota = jax.lax.broadcasted_iota(jnp.int32, (S, S), 1)
    ut = (t_iota <= s_iota).astype(f32)
    cum = jnp.dot(mf, ut)
    order = jnp.clip(cum - 1.0, 0.0, float(E - 1))
    bi = jax.lax.broadcasted_iota(jnp.int32, (B, S), 0).astype(f32)
    idxf = jnp.where(mask, order + bi * float(E), float(B * E))
    idx_ref[...] = idxf.astype(jnp.int32)


def _main_body(q2_ref, q1_ref, k2_ref, v2_ref, k1_ref, v1_ref, c1_ref):
    f32 = jnp.float32
    bidx = pl.program_id(0)

    @pl.when(bidx < B)
    def _compute():
        k2 = k2_ref[0]
        v2 = v2_ref[0]
        q2 = q2_ref[0][None, :, :, None]
        l2 = jnp.sum(k2 * q2, axis=1)
        e2 = jnp.exp(l2)
        attn2 = e2 / jnp.sum(e2, axis=1, keepdims=True)
        comb = jnp.sum(attn2[:, None, :, :] * v2, axis=2)

        k1 = k1_ref[0]
        v1 = v1_ref[0]
        q1 = q1_ref[0][:, :, None]
        l1 = jnp.sum(k1 * q1, axis=0)
        e1 = jnp.exp(l1)
        attn1 = e1 / jnp.sum(e1, axis=0, keepdims=True)
        o_a = jnp.sum(attn1[None, :, :] * v1, axis=1)
        o_b = jnp.sum(attn1[:, None, :] * comb, axis=0)
        c1 = jnp.concatenate([o_a, o_b], axis=0)
        c1t = jnp.transpose(c1)
        c1_ref[...] = jnp.pad(c1t, ((0, 0), (0, C1W - 2 * KV)))

    @pl.when(bidx == B)
    def _zeros():
        c1_ref[...] = jnp.zeros((E, C1W), f32)


def _sc_gather(c1r, idx):
    vmesh = plsc.VectorSubcoreMesh(core_axis_name="core",
                                   subcore_axis_name="subcore")

    @pl.kernel(out_type=jax.ShapeDtypeStruct((B * S, C1W), c1r.dtype),
               mesh=vmesh)
    def gather_kernel(c1_hbm, i_hbm, o_hbm):
        def body(i_vmem, o_vmem):
            pltpu.sync_copy(c1_hbm.at[i_vmem.at[0]], o_vmem)

        pltpu.emit_pipeline(
            body,
            grid=(B * S // GW,),
            in_specs=[pl.BlockSpec((1, GW), index_map=lambda i: (0, i))],
            out_specs=[pl.BlockSpec((GW, C1W), index_map=lambda i: (i, 0))],
            core_axis_name=("core", "subcore"),
            dimension_semantics=(pltpu.PARALLEL,),
        )(i_hbm, o_hbm)

    return gather_kernel(c1r, idx)


@jax.jit
def _run(input_ent, q, k_hop1, v_hop1, k_hop2, v_hop2, w_q2, b_q2, w_k2,
         w_q1, b_q1, w_k1):
    f32 = jnp.float32
    q0 = q[:, 0, :]
    ient = input_ent.astype(jnp.int32)
    qc2, qc1, idx = pl.pallas_call(
        _prep_body,
        out_shape=(jax.ShapeDtypeStruct((B, KV, 1), f32),
                   jax.ShapeDtypeStruct((B, KV, 1), f32),
                   jax.ShapeDtypeStruct((B, S), jnp.int32)),
    )(ient, q0, w_q2, b_q2.reshape(1, KV), w_k2, w_q1,
      b_q1.reshape(1, KV), w_k1)

    k2t = jnp.transpose(k_hop2, (0, 2, 4, 3, 1))
    v2t = jnp.transpose(v_hop2, (0, 2, 4, 3, 1))
    k1t = jnp.transpose(k_hop1, (0, 3, 2, 1))
    v1t = jnp.transpose(v_hop1, (0, 3, 2, 1))

    cb = lambda b: jnp.minimum(b, B - 1)
    c1r = pl.pallas_call(
        _main_body,
        grid=(B + 1,),
        in_specs=[
            pl.BlockSpec((1, KV, 1), lambda b: (cb(b), 0, 0)),
            pl.BlockSpec((1, KV, 1), lambda b: (cb(b), 0, 0)),
            pl.BlockSpec((1, N1, KV, N2, E), lambda b: (cb(b), 0, 0, 0, 0)),
            pl.BlockSpec((1, N1, KV, N2, E), lambda b: (cb(b), 0, 0, 0, 0)),
            pl.BlockSpec((1, KV, N1, E), lambda b: (cb(b), 0, 0, 0)),
            pl.BlockSpec((1, KV, N1, E), lambda b: (cb(b), 0, 0, 0)),
        ],
        out_specs=pl.BlockSpec((E, C1W), lambda b: (b, 0)),
        out_shape=jax.ShapeDtypeStruct(((B + 1) * E, C1W), f32),
        compiler_params=pltpu.CompilerParams(
            dimension_semantics=("arbitrary",),
        ),
    )(qc2, qc1, k2t, v2t, k1t, v1t)

    out_flat = _sc_gather(c1r, idx.reshape(1, B * S))
    return out_flat[:, :2 * KV].reshape(B, S, 2 * KV)


def kernel(input_ent, q, k_hop1, v_hop1, k_hop2, v_hop2, w_q2, b_q2, w_k2,
           w_q1, b_q1, w_k1):
    return _run(input_ent, q, k_hop1, v_hop1, k_hop2, v_hop2, w_q2, b_q2,
                w_k2, w_q1, b_q1, w_k1)
